# Initial kernel scaffold; baseline (speedup 1.0000x reference)
import jax
import jax.numpy as jnp
from jax.experimental import pallas as pl


def kernel(adj, W_lin, b_lin, W_self0, W_neigh0, b0, W_res0, b_res0, W_self1, W_neigh1, b1, W_res1, b_res1):
    raise NotImplementedError("write your pallas kernel here")


# speedup vs baseline: 1.7337x; 1.7337x over previous
import functools

import jax
import jax.numpy as jnp
from jax.experimental import pallas as pl


def _sage_layer_body(adj_ref, h_ref, ws_ref, wn_ref, b_ref, wr_ref, br_ref,
                     out_ref, *, block_rows, with_act):
    i = pl.program_id(0)
    a = adj_ref[...]
    h = h_ref[...]
    deg = jnp.sum(a, axis=1, keepdims=True)
    agg = jnp.dot(a, h, preferred_element_type=jnp.float32)
    neigh = agg / jnp.clip(deg, 1e-6, None)
    hblk = h_ref[pl.ds(i * block_rows, block_rows), :]
    m = (jnp.dot(hblk, ws_ref[...], preferred_element_type=jnp.float32)
         + jnp.dot(neigh, wn_ref[...], preferred_element_type=jnp.float32)
         + b_ref[...])
    if with_act:
        m = jnp.where(m >= 0, m, 0.01 * m)
    out_ref[...] = (m
                    + jnp.dot(hblk, wr_ref[...], preferred_element_type=jnp.float32)
                    + br_ref[...])


def _pick_block_rows(n):
    for b in (400, 200, 80, 40, 16, 8):
        if n % b == 0:
            return b
    return n


def _sage_layer(adj, h, ws, wn, b, wr, br, with_act):
    n, d_in = h.shape
    d_out = ws.shape[1]
    block_rows = _pick_block_rows(n)
    body = functools.partial(_sage_layer_body, block_rows=block_rows,
                             with_act=with_act)
    return pl.pallas_call(
        body,
        grid=(n // block_rows,),
        in_specs=[
            pl.BlockSpec((block_rows, n), lambda i: (i, 0)),
            pl.BlockSpec((n, d_in), lambda i: (0, 0)),
            pl.BlockSpec((d_in, d_out), lambda i: (0, 0)),
            pl.BlockSpec((d_in, d_out), lambda i: (0, 0)),
            pl.BlockSpec((1, d_out), lambda i: (0, 0)),
            pl.BlockSpec((d_in, d_out), lambda i: (0, 0)),
            pl.BlockSpec((1, d_out), lambda i: (0, 0)),
        ],
        out_specs=pl.BlockSpec((block_rows, d_out), lambda i: (i, 0)),
        out_shape=jax.ShapeDtypeStruct((n, d_out), jnp.float32),
    )(adj, h, ws, wn, b.reshape(1, -1), wr, br.reshape(1, -1))


def kernel(adj, W_lin, b_lin, W_self0, W_neigh0, b0, W_res0, b_res0,
           W_self1, W_neigh1, b1, W_res1, b_res1):
    h0 = W_lin + b_lin[None, :]
    h1 = _sage_layer(adj, h0, W_self0, W_neigh0, b0, W_res0, b_res0,
                     with_act=True)
    out = _sage_layer(adj, h1, W_self1, W_neigh1, b1, W_res1, b_res1,
                      with_act=False)
    return (out, h1, out)
